# baseline (speedup 1.0000x reference)
import functools

import jax
import jax.numpy as jnp
from jax import lax
from jax.experimental import pallas as pl
from jax.experimental.pallas import tpu as pltpu
from jax.experimental.pallas import tpu_sc as plsc

BS = 1024
D = 64
NN = 501
NPAD = 512
NC = 2
NS = 16
NW = NC * NS
BPW = BS // NW
NCHUNK = 128
L = 16


def _sincos_body(day_ref, w1_ref, w2_ref, dreal_ref, dimg_ref):
    ang1 = w1_ref[:] * day_ref[:]
    ang2 = w2_ref[:] * day_ref[:]
    dimg_ref[:] = jnp.sin(ang1)
    dreal_ref[:] = jnp.cos(ang2)


def _sincos(day, w1, w2):
    return pl.pallas_call(
        _sincos_body,
        out_shape=(
            jax.ShapeDtypeStruct((BS, D), jnp.float32),
            jax.ShapeDtypeStruct((BS, D), jnp.float32),
        ),
    )(day.reshape(BS, 1), w1.reshape(1, D), w2.reshape(1, D))


def _pack_body(tr_ref, ti_ref, out_ref):
    out_ref[:, 0:D] = tr_ref[:].T
    out_ref[:, D:2 * D] = ti_ref[:].T


def _pack(eR, eI, n_rows, bw):
    return pl.pallas_call(
        _pack_body,
        grid=(pl.cdiv(n_rows, bw),),
        in_specs=[
            pl.BlockSpec((D, bw), lambda i: (0, i)),
            pl.BlockSpec((D, bw), lambda i: (0, i)),
        ],
        out_specs=pl.BlockSpec((bw, 2 * D), lambda i: (i, 0)),
        out_shape=jax.ShapeDtypeStruct((n_rows, 2 * D), jnp.float32),
    )(eR.T, eI.T)


def _ce_body(scores_ref, out_ref):
    s = scores_ref[:]
    col = lax.broadcasted_iota(jnp.int32, (BS, NPAD), 1)
    s = jnp.where(col < NN, s, -1e30)
    m = jnp.max(s, axis=1, keepdims=True)
    lse = m[:, 0] + jnp.log(jnp.sum(jnp.exp(s - m), axis=1))
    loss = lse - s[:, 0]
    out_ref[0, 0] = jnp.sum(loss) * (1.0 / BS)


def _ce(scores):
    return pl.pallas_call(
        _ce_body,
        out_shape=jax.ShapeDtypeStruct((1, 1), jnp.float32),
        out_specs=pl.BlockSpec(memory_space=pltpu.SMEM),
    )(scores)


def _sc_scores_body(sub_hbm, rel_hbm, ent_hbm, dr_hbm, di_hbm,
                    tabE_hbm, tabR_hbm, out_hbm,
                    sub_v, rel_v, esub_v, rrow_v, dc_v, ds_v,
                    idxall_v, rows0_v, rows1_v, rows2_v, rows3_v, scores_v,
                    sem0, sem1, sem2, sem3):
    wid = lax.axis_index("s") * NC + lax.axis_index("c")
    base = wid * BPW

    pltpu.sync_copy(sub_hbm.at[pl.ds(base, BPW)], sub_v)
    pltpu.sync_copy(rel_hbm.at[pl.ds(base, BPW)], rel_v)
    pltpu.sync_copy(dr_hbm.at[pl.ds(base, BPW)], dc_v)
    pltpu.sync_copy(di_hbm.at[pl.ds(base, BPW)], ds_v)
    pltpu.sync_copy(ent_hbm.at[pl.ds(base * NPAD, BPW * NPAD)], idxall_v)
    c1 = pltpu.async_copy(tabE_hbm.at[sub_v], esub_v, sem0)
    c2 = pltpu.async_copy(tabR_hbm.at[rel_v], rrow_v, sem1)
    c1.wait()
    c2.wait()

    lane = lax.iota(jnp.int32, L)
    dnums = lax.GatherDimensionNumbers(
        offset_dims=(), collapsed_slice_dims=(0,), start_index_map=(0,))

    def _shuf_xor(v, k):
        p = jnp.bitwise_xor(lane, k)
        return lax.gather(v, p[:, None], dnums, (1,),
                          mode=lax.GatherScatterMode.PROMISE_IN_BOUNDS)

    def _tree_reduce(accs):
        k = 1
        while len(accs) > 1:
            mask = (lane & k) != 0
            nxt = []
            for i in range(0, len(accs), 2):
                a, b = accs[i], accs[i + 1]
                own = jnp.where(mask, b, a)
                oth = jnp.where(mask, a, b)
                nxt.append(own + _shuf_xor(oth, k))
            accs = nxt
            k *= 2
        return accs[0]

    rows = (rows0_v, rows1_v, rows2_v, rows3_v)
    sems = (sem0, sem1, sem2, sem3)
    NB = 4
    CPB = NPAD // NCHUNK
    K = BPW * CPB

    NSPLIT = 4
    QR = NCHUNK // NSPLIT

    def _issue(j, par):
        for q in range(NSPLIT):
            pltpu.async_copy(
                tabE_hbm.at[idxall_v.at[pl.ds(j * NCHUNK + q * QR, QR)]],
                rows[par].at[pl.ds(q * QR, QR)], sems[par])

    for par in range(NB):
        _issue(par, par)

    def bi_body(bi, _):
        arc = []
        aic = []
        dcc = []
        dsc = []
        for c in range(4):
            sl = pl.ds(c * L, L)
            isl = pl.ds(D + c * L, L)
            esr = esub_v[bi, sl]
            esi = esub_v[bi, isl]
            dc = dc_v[bi, sl]
            dsn = ds_v[bi, sl]
            arc.append(esr * dc - esi * dsn + rrow_v[bi, sl])
            aic.append(esr * dsn + esi * dc + rrow_v[bi, isl])
            dcc.append(dc)
            dsc.append(dsn)

        for nc in range(CPB):
            k = bi * CPB + nc
            rv = rows[nc]
            pltpu.make_async_copy(
                tabE_hbm.at[idxall_v.at[pl.ds(0, NCHUNK)]], rv, sems[nc]
            ).wait()

            def g_body(g, _g):
                n0 = g * L
                accs = []
                for j in range(L):
                    acc = None
                    for c in range(4):
                        sl = pl.ds(c * L, L)
                        er = rv[n0 + j, sl]
                        ei = rv[n0 + j, pl.ds(D + c * L, L)]
                        tr = er * dcc[c] - ei * dsc[c]
                        ti = er * dsc[c] + ei * dcc[c]
                        t = jnp.abs(arc[c] - tr) + jnp.abs(aic[c] + ti)
                        acc = t if acc is None else acc + t
                    accs.append(acc)
                scores_v[pl.ds(nc * NCHUNK + g * L, L)] = _tree_reduce(accs)
                return 0

            lax.fori_loop(0, NCHUNK // L, g_body, 0)

            @pl.when(k + NB < K)
            def _():
                _issue(k + NB, nc)

        pltpu.sync_copy(scores_v, out_hbm.at[base + bi])
        return 0

    lax.fori_loop(0, BPW, bi_body, 0)


@functools.cache
def _build_sc_scores():
    return functools.partial(
        pl.kernel,
        mesh=plsc.VectorSubcoreMesh(core_axis_name="c", subcore_axis_name="s"),
        out_type=jax.ShapeDtypeStruct((BS, NPAD), jnp.float32),
        compiler_params=pltpu.CompilerParams(use_tc_tiling_on_sc=False),
        scratch_types=[
            pltpu.VMEM((BPW,), jnp.int32),
            pltpu.VMEM((BPW,), jnp.int32),
            pltpu.VMEM((BPW, 2 * D), jnp.float32),
            pltpu.VMEM((BPW, 2 * D), jnp.float32),
            pltpu.VMEM((BPW, D), jnp.float32),
            pltpu.VMEM((BPW, D), jnp.float32),
            pltpu.VMEM((BPW * NPAD,), jnp.int32),
            pltpu.VMEM((NCHUNK, 2 * D), jnp.float32),
            pltpu.VMEM((NCHUNK, 2 * D), jnp.float32),
            pltpu.VMEM((NCHUNK, 2 * D), jnp.float32),
            pltpu.VMEM((NCHUNK, 2 * D), jnp.float32),
            pltpu.VMEM((NPAD,), jnp.float32),
            pltpu.SemaphoreType.DMA,
            pltpu.SemaphoreType.DMA,
            pltpu.SemaphoreType.DMA,
            pltpu.SemaphoreType.DMA,
        ],
    )(_sc_scores_body)


def kernel(sub, rel, obj, year, month, day, neg, emb_E_real, emb_E_img,
           emb_R_real, emb_R_img, w1, w2):
    del year, month
    ent = jnp.concatenate([obj[:, None], neg], axis=1).astype(jnp.int32)
    ent = jnp.pad(ent, ((0, 0), (0, NPAD - NN)))
    ent_flat = ent.reshape(-1)
    d_real, d_img = _sincos(day, w1, w2)
    tabE = _pack(emb_E_real, emb_E_img, 1000000, 8192)
    tabR = _pack(emb_R_real, emb_R_img, 1000, 1000)
    scores = _build_sc_scores()(sub.astype(jnp.int32), rel.astype(jnp.int32),
                                ent_flat, d_real, d_img, tabE, tabR)
    return _ce(scores)[0, 0]


# speedup vs baseline: 1.0042x; 1.0042x over previous
import functools

import jax
import jax.numpy as jnp
from jax import lax
from jax.experimental import pallas as pl
from jax.experimental.pallas import tpu as pltpu
from jax.experimental.pallas import tpu_sc as plsc

BS = 1024
D = 64
NN = 501
NPAD = 512
NC = 2
NS = 16
NW = NC * NS
BPW = BS // NW
NCHUNK = 128
L = 16


def _sincos_body(day_ref, w1_ref, w2_ref, dreal_ref, dimg_ref):
    ang1 = w1_ref[:] * day_ref[:]
    ang2 = w2_ref[:] * day_ref[:]
    dimg_ref[:] = jnp.sin(ang1)
    dreal_ref[:] = jnp.cos(ang2)


def _sincos(day, w1, w2):
    return pl.pallas_call(
        _sincos_body,
        out_shape=(
            jax.ShapeDtypeStruct((BS, D), jnp.float32),
            jax.ShapeDtypeStruct((BS, D), jnp.float32),
        ),
    )(day.reshape(BS, 1), w1.reshape(1, D), w2.reshape(1, D))


def _pack_body(tr_ref, ti_ref, out_ref):
    out_ref[:, 0:D] = tr_ref[:].T
    out_ref[:, D:2 * D] = ti_ref[:].T


def _pack(eR, eI, n_rows, bw):
    return pl.pallas_call(
        _pack_body,
        grid=(pl.cdiv(n_rows, bw),),
        in_specs=[
            pl.BlockSpec((D, bw), lambda i: (0, i)),
            pl.BlockSpec((D, bw), lambda i: (0, i)),
        ],
        out_specs=pl.BlockSpec((bw, 2 * D), lambda i: (i, 0)),
        out_shape=jax.ShapeDtypeStruct((n_rows, 2 * D), jnp.float32),
    )(eR.T, eI.T)


def _ce_body(scores_ref, out_ref):
    s = scores_ref[:]
    col = lax.broadcasted_iota(jnp.int32, (BS, NPAD), 1)
    s = jnp.where(col < NN, s, -1e30)
    m = jnp.max(s, axis=1, keepdims=True)
    lse = m[:, 0] + jnp.log(jnp.sum(jnp.exp(s - m), axis=1))
    loss = lse - s[:, 0]
    out_ref[0, 0] = jnp.sum(loss) * (1.0 / BS)


def _ce(scores):
    return pl.pallas_call(
        _ce_body,
        out_shape=jax.ShapeDtypeStruct((1, 1), jnp.float32),
        out_specs=pl.BlockSpec(memory_space=pltpu.SMEM),
    )(scores)


def _sc_scores_body(sub_hbm, rel_hbm, ent_hbm, dr_hbm, di_hbm,
                    tabE_hbm, tabR_hbm, out_hbm,
                    sub_v, rel_v, i2a_v, i2b_v, esr_v, esi_v, rr_v, ri_v,
                    dc_v, ds_v, idxall_v,
                    ir0_v, ii0_v, ir1_v, ii1_v,
                    er0_v, ei0_v, er1_v, ei1_v, scores_v,
                    sem0, sem1, sem2, sem3):
    wid = lax.axis_index("s") * NC + lax.axis_index("c")
    base = wid * BPW

    pltpu.sync_copy(sub_hbm.at[pl.ds(base, BPW)], sub_v)
    pltpu.sync_copy(rel_hbm.at[pl.ds(base, BPW)], rel_v)
    pltpu.sync_copy(dr_hbm.at[pl.ds(base, BPW)], dc_v)
    pltpu.sync_copy(di_hbm.at[pl.ds(base, BPW)], ds_v)
    pltpu.sync_copy(ent_hbm.at[pl.ds(base * NPAD, BPW * NPAD)], idxall_v)
    for h in range(BPW // L):
        s16 = sub_v[pl.ds(h * L, L)]
        i2a_v[pl.ds(h * L, L)] = s16 * 2
        i2b_v[pl.ds(h * L, L)] = s16 * 2 + 1
    c1 = pltpu.async_copy(tabE_hbm.at[i2a_v], esr_v, sem0)
    c2 = pltpu.async_copy(tabE_hbm.at[i2b_v], esi_v, sem1)
    c1.wait()
    c2.wait()
    for h in range(BPW // L):
        s16 = rel_v[pl.ds(h * L, L)]
        i2a_v[pl.ds(h * L, L)] = s16 * 2
        i2b_v[pl.ds(h * L, L)] = s16 * 2 + 1
    c1 = pltpu.async_copy(tabR_hbm.at[i2a_v], rr_v, sem0)
    c2 = pltpu.async_copy(tabR_hbm.at[i2b_v], ri_v, sem1)
    c1.wait()
    c2.wait()

    lane = lax.iota(jnp.int32, L)
    dnums = lax.GatherDimensionNumbers(
        offset_dims=(), collapsed_slice_dims=(0,), start_index_map=(0,))

    def _shuf_xor(v, k):
        p = jnp.bitwise_xor(lane, k)
        return lax.gather(v, p[:, None], dnums, (1,),
                          mode=lax.GatherScatterMode.PROMISE_IN_BOUNDS)

    def _tree_reduce(accs):
        k = 1
        while len(accs) > 1:
            mask = (lane & k) != 0
            nxt = []
            for i in range(0, len(accs), 2):
                a, b = accs[i], accs[i + 1]
                own = jnp.where(mask, b, a)
                oth = jnp.where(mask, a, b)
                nxt.append(own + _shuf_xor(oth, k))
            accs = nxt
            k *= 2
        return accs[0]

    irs = (ir0_v, ir1_v)
    iis = (ii0_v, ii1_v)
    ers = (er0_v, er1_v)
    eis = (ei0_v, ei1_v)
    rsem = (sem0, sem1)
    isem = (sem2, sem3)
    NB = 2
    CPB = NPAD // NCHUNK
    K = BPW * CPB

    def _issue(j, slot):
        for h in range(NCHUNK // L):
            e16 = idxall_v[pl.ds(j * NCHUNK + h * L, L)]
            irs[slot][pl.ds(h * L, L)] = e16 * 2
            iis[slot][pl.ds(h * L, L)] = e16 * 2 + 1
        pltpu.async_copy(tabE_hbm.at[irs[slot]], ers[slot], rsem[slot])
        pltpu.async_copy(tabE_hbm.at[iis[slot]], eis[slot], isem[slot])

    _issue(0, 0)
    _issue(1, 1)

    def bi_body(bi, _):
        arc = []
        aic = []
        dcc = []
        dsc = []
        for c in range(4):
            sl = pl.ds(c * L, L)
            esr = esr_v[bi, sl]
            esi = esi_v[bi, sl]
            dc = dc_v[bi, sl]
            dsn = ds_v[bi, sl]
            arc.append(esr * dc - esi * dsn + rr_v[bi, sl])
            aic.append(esr * dsn + esi * dc + ri_v[bi, sl])
            dcc.append(dc)
            dsc.append(dsn)

        for nc in range(CPB):
            k = bi * CPB + nc
            slot = nc % NB
            erv = ers[slot]
            eiv = eis[slot]
            pltpu.make_async_copy(
                tabE_hbm.at[irs[slot]], erv, rsem[slot]).wait()
            pltpu.make_async_copy(
                tabE_hbm.at[iis[slot]], eiv, isem[slot]).wait()

            def g_body(g, _g):
                n0 = g * L
                accs = []
                for j in range(L):
                    acc = None
                    for c in range(4):
                        sl = pl.ds(c * L, L)
                        er = erv[n0 + j, sl]
                        ei = eiv[n0 + j, sl]
                        tr = er * dcc[c] - ei * dsc[c]
                        ti = er * dsc[c] + ei * dcc[c]
                        t = jnp.abs(arc[c] - tr) + jnp.abs(aic[c] + ti)
                        acc = t if acc is None else acc + t
                    accs.append(acc)
                scores_v[pl.ds(nc * NCHUNK + g * L, L)] = _tree_reduce(accs)
                return 0

            lax.fori_loop(0, NCHUNK // L, g_body, 0)

            @pl.when(k + NB < K)
            def _():
                _issue(k + NB, slot)

        pltpu.sync_copy(scores_v, out_hbm.at[base + bi])
        return 0

    lax.fori_loop(0, BPW, bi_body, 0)


@functools.cache
def _build_sc_scores():
    return functools.partial(
        pl.kernel,
        mesh=plsc.VectorSubcoreMesh(core_axis_name="c", subcore_axis_name="s"),
        out_type=jax.ShapeDtypeStruct((BS, NPAD), jnp.float32),
        compiler_params=pltpu.CompilerParams(use_tc_tiling_on_sc=False),
        scratch_types=[
            pltpu.VMEM((BPW,), jnp.int32),
            pltpu.VMEM((BPW,), jnp.int32),
            pltpu.VMEM((BPW,), jnp.int32),
            pltpu.VMEM((BPW,), jnp.int32),
            pltpu.VMEM((BPW, D), jnp.float32),
            pltpu.VMEM((BPW, D), jnp.float32),
            pltpu.VMEM((BPW, D), jnp.float32),
            pltpu.VMEM((BPW, D), jnp.float32),
            pltpu.VMEM((BPW, D), jnp.float32),
            pltpu.VMEM((BPW, D), jnp.float32),
            pltpu.VMEM((BPW * NPAD,), jnp.int32),
            pltpu.VMEM((NCHUNK,), jnp.int32),
            pltpu.VMEM((NCHUNK,), jnp.int32),
            pltpu.VMEM((NCHUNK,), jnp.int32),
            pltpu.VMEM((NCHUNK,), jnp.int32),
            pltpu.VMEM((NCHUNK, D), jnp.float32),
            pltpu.VMEM((NCHUNK, D), jnp.float32),
            pltpu.VMEM((NCHUNK, D), jnp.float32),
            pltpu.VMEM((NCHUNK, D), jnp.float32),
            pltpu.VMEM((NPAD,), jnp.float32),
            pltpu.SemaphoreType.DMA,
            pltpu.SemaphoreType.DMA,
            pltpu.SemaphoreType.DMA,
            pltpu.SemaphoreType.DMA,
        ],
    )(_sc_scores_body)


def kernel(sub, rel, obj, year, month, day, neg, emb_E_real, emb_E_img,
           emb_R_real, emb_R_img, w1, w2):
    del year, month
    ent = jnp.concatenate([obj[:, None], neg], axis=1).astype(jnp.int32)
    ent = jnp.pad(ent, ((0, 0), (0, NPAD - NN)))
    ent_flat = ent.reshape(-1)
    d_real, d_img = _sincos(day, w1, w2)
    tabE = _pack(emb_E_real, emb_E_img, 1000000, 8192).reshape(2000000, D)
    tabR = _pack(emb_R_real, emb_R_img, 1000, 1000).reshape(2000, D)
    scores = _build_sc_scores()(sub.astype(jnp.int32), rel.astype(jnp.int32),
                                ent_flat, d_real, d_img, tabE, tabR)
    return _ce(scores)[0, 0]


# speedup vs baseline: 1.1611x; 1.1562x over previous
import functools

import jax
import jax.numpy as jnp
from jax import lax
from jax.experimental import pallas as pl
from jax.experimental.pallas import tpu as pltpu
from jax.experimental.pallas import tpu_sc as plsc

BS = 1024
D = 64
NN = 501
NPAD = 512
NC = 2
NS = 16
NW = NC * NS
BPW = BS // NW
NCHUNK = 128
L = 16


def _sincos_body(day_ref, w1_ref, w2_ref, dreal_ref, dimg_ref):
    ang1 = w1_ref[:] * day_ref[:]
    ang2 = w2_ref[:] * day_ref[:]
    dimg_ref[:] = jnp.sin(ang1)
    dreal_ref[:] = jnp.cos(ang2)


def _sincos(day, w1, w2):
    return pl.pallas_call(
        _sincos_body,
        out_shape=(
            jax.ShapeDtypeStruct((BS, D), jnp.float32),
            jax.ShapeDtypeStruct((BS, D), jnp.float32),
        ),
    )(day.reshape(BS, 1), w1.reshape(1, D), w2.reshape(1, D))


PBW = 8192
NEPAD = 1 << 20


def _pack_body(tr_ref, out_ref):
    t = tr_ref[:]
    h = t.shape[1] // 2
    out_ref[:, 0:D] = t[:, 0:h].T
    out_ref[:, D:2 * D] = t[:, h:].T


def _pack(eT, n_rows, n_pad, bw):
    out = pl.pallas_call(
        _pack_body,
        grid=(pl.cdiv(n_rows, bw),),
        in_specs=[pl.BlockSpec((D, bw), lambda i: (0, i))],
        out_specs=pl.BlockSpec((bw // 2, 2 * D), lambda i: (i, 0)),
        out_shape=jax.ShapeDtypeStruct((n_pad // 2, 2 * D), jnp.float32),
    )(eT.T)
    return out.reshape(n_pad, D)


def _ce_body(scores_ref, out_ref):
    s = scores_ref[:]
    col = lax.broadcasted_iota(jnp.int32, (BS, NPAD), 1)
    s = jnp.where(col < NN, s, -1e30)
    m = jnp.max(s, axis=1, keepdims=True)
    lse = m[:, 0] + jnp.log(jnp.sum(jnp.exp(s - m), axis=1))
    loss = lse - s[:, 0]
    out_ref[0, 0] = jnp.sum(loss) * (1.0 / BS)


def _ce(scores):
    return pl.pallas_call(
        _ce_body,
        out_shape=jax.ShapeDtypeStruct((1, 1), jnp.float32),
        out_specs=pl.BlockSpec(memory_space=pltpu.SMEM),
    )(scores)


def _sc_scores_body(sub_hbm, rel_hbm, ent_hbm, dr_hbm, di_hbm,
                    tabER_hbm, tabEI_hbm, tabRR_hbm, tabRI_hbm, out_hbm,
                    sub_v, rel_v, esr_v, esi_v, rr_v, ri_v,
                    dc_v, ds_v, idxall_v, ix0_v, ix1_v,
                    er0_v, ei0_v, er1_v, ei1_v, scores_v,
                    sem0, sem1, sem2, sem3):
    wid = lax.axis_index("s") * NC + lax.axis_index("c")
    base = wid * BPW

    def _sigma_e(e):
        j = e & (PBW - 1)
        return e - j + 2 * (j & (PBW // 2 - 1)) + (j >> 12)

    def _sigma_r(e):
        return jnp.where(e >= 500, 2 * e - 999, 2 * e)

    pltpu.sync_copy(sub_hbm.at[pl.ds(base, BPW)], sub_v)
    pltpu.sync_copy(rel_hbm.at[pl.ds(base, BPW)], rel_v)
    pltpu.sync_copy(dr_hbm.at[pl.ds(base, BPW)], dc_v)
    pltpu.sync_copy(di_hbm.at[pl.ds(base, BPW)], ds_v)
    pltpu.sync_copy(ent_hbm.at[pl.ds(base * NPAD, BPW * NPAD)], idxall_v)
    for h in range(BPW // L):
        sl = pl.ds(h * L, L)
        ix0_v[sl] = _sigma_e(sub_v[sl])
        ix1_v[sl] = _sigma_r(rel_v[sl])
    sub_ix = ix0_v.at[pl.ds(0, BPW)]
    rel_ix = ix1_v.at[pl.ds(0, BPW)]
    c1 = pltpu.async_copy(tabER_hbm.at[sub_ix], esr_v, sem0)
    c2 = pltpu.async_copy(tabEI_hbm.at[sub_ix], esi_v, sem1)
    c1.wait()
    c2.wait()
    c3 = pltpu.async_copy(tabRR_hbm.at[rel_ix], rr_v, sem0)
    c4 = pltpu.async_copy(tabRI_hbm.at[rel_ix], ri_v, sem1)
    c3.wait()
    c4.wait()

    lane = lax.iota(jnp.int32, L)
    dnums = lax.GatherDimensionNumbers(
        offset_dims=(), collapsed_slice_dims=(0,), start_index_map=(0,))

    def _shuf_xor(v, k):
        p = jnp.bitwise_xor(lane, k)
        return lax.gather(v, p[:, None], dnums, (1,),
                          mode=lax.GatherScatterMode.PROMISE_IN_BOUNDS)

    def _tree_reduce(accs):
        k = 1
        while len(accs) > 1:
            mask = (lane & k) != 0
            nxt = []
            for i in range(0, len(accs), 2):
                a, b = accs[i], accs[i + 1]
                own = jnp.where(mask, b, a)
                oth = jnp.where(mask, a, b)
                nxt.append(own + _shuf_xor(oth, k))
            accs = nxt
            k *= 2
        return accs[0]

    ers = (er0_v, er1_v)
    eis = (ei0_v, ei1_v)
    rsem = (sem0, sem1)
    isem = (sem2, sem3)
    NB = 2
    CPB = NPAD // NCHUNK
    K = BPW * CPB

    ixs = (ix0_v, ix1_v)

    def _issue(j, slot):
        for h in range(NCHUNK // L):
            e16 = idxall_v[pl.ds(j * NCHUNK + h * L, L)]
            ixs[slot][pl.ds(h * L, L)] = _sigma_e(e16)
        pltpu.async_copy(tabER_hbm.at[ixs[slot]], ers[slot], rsem[slot])
        pltpu.async_copy(tabEI_hbm.at[ixs[slot]], eis[slot], isem[slot])

    _issue(0, 0)
    _issue(1, 1)

    def bi_body(bi, _):
        arc = []
        aic = []
        dcc = []
        dsc = []
        for c in range(4):
            sl = pl.ds(c * L, L)
            esr = esr_v[bi, sl]
            esi = esi_v[bi, sl]
            dc = dc_v[bi, sl]
            dsn = ds_v[bi, sl]
            arc.append(esr * dc - esi * dsn + rr_v[bi, sl])
            aic.append(esr * dsn + esi * dc + ri_v[bi, sl])
            dcc.append(dc)
            dsc.append(dsn)

        for nc in range(CPB):
            k = bi * CPB + nc
            slot = nc % NB
            erv = ers[slot]
            eiv = eis[slot]
            pltpu.make_async_copy(
                tabER_hbm.at[ixs[slot]], erv, rsem[slot]).wait()
            pltpu.make_async_copy(
                tabEI_hbm.at[ixs[slot]], eiv, isem[slot]).wait()

            def g_body(g, _g):
                n0 = g * L
                accs = []
                for j in range(L):
                    acc = None
                    for c in range(4):
                        sl = pl.ds(c * L, L)
                        er = erv[n0 + j, sl]
                        ei = eiv[n0 + j, sl]
                        tr = er * dcc[c] - ei * dsc[c]
                        ti = er * dsc[c] + ei * dcc[c]
                        t = jnp.abs(arc[c] - tr) + jnp.abs(aic[c] + ti)
                        acc = t if acc is None else acc + t
                    accs.append(acc)
                scores_v[pl.ds(nc * NCHUNK + g * L, L)] = _tree_reduce(accs)
                return 0

            lax.fori_loop(0, NCHUNK // L, g_body, 0)

            @pl.when(k + NB < K)
            def _():
                _issue(k + NB, slot)

        pltpu.sync_copy(scores_v, out_hbm.at[base + bi])
        return 0

    lax.fori_loop(0, BPW, bi_body, 0)


@functools.cache
def _build_sc_scores():
    return functools.partial(
        pl.kernel,
        mesh=plsc.VectorSubcoreMesh(core_axis_name="c", subcore_axis_name="s"),
        out_type=jax.ShapeDtypeStruct((BS, NPAD), jnp.float32),
        compiler_params=pltpu.CompilerParams(use_tc_tiling_on_sc=False),
        scratch_types=[
            pltpu.VMEM((BPW,), jnp.int32),
            pltpu.VMEM((BPW,), jnp.int32),
            pltpu.VMEM((BPW, D), jnp.float32),
            pltpu.VMEM((BPW, D), jnp.float32),
            pltpu.VMEM((BPW, D), jnp.float32),
            pltpu.VMEM((BPW, D), jnp.float32),
            pltpu.VMEM((BPW, D), jnp.float32),
            pltpu.VMEM((BPW, D), jnp.float32),
            pltpu.VMEM((BPW * NPAD,), jnp.int32),
            pltpu.VMEM((NCHUNK,), jnp.int32),
            pltpu.VMEM((NCHUNK,), jnp.int32),
            pltpu.VMEM((NCHUNK, D), jnp.float32),
            pltpu.VMEM((NCHUNK, D), jnp.float32),
            pltpu.VMEM((NCHUNK, D), jnp.float32),
            pltpu.VMEM((NCHUNK, D), jnp.float32),
            pltpu.VMEM((NPAD,), jnp.float32),
            pltpu.SemaphoreType.DMA,
            pltpu.SemaphoreType.DMA,
            pltpu.SemaphoreType.DMA,
            pltpu.SemaphoreType.DMA,
        ],
    )(_sc_scores_body)


def kernel(sub, rel, obj, year, month, day, neg, emb_E_real, emb_E_img,
           emb_R_real, emb_R_img, w1, w2):
    del year, month
    ent = jnp.concatenate([obj[:, None], neg], axis=1).astype(jnp.int32)
    ent = jnp.pad(ent, ((0, 0), (0, NPAD - NN)))
    ent_flat = ent.reshape(-1)
    d_real, d_img = _sincos(day, w1, w2)
    tabER = _pack(emb_E_real, 1000000, NEPAD, PBW)
    tabEI = _pack(emb_E_img, 1000000, NEPAD, PBW)
    tabRR = _pack(emb_R_real, 1000, 1000, 1000)
    tabRI = _pack(emb_R_img, 1000, 1000, 1000)
    scores = _build_sc_scores()(sub.astype(jnp.int32), rel.astype(jnp.int32),
                                ent_flat, d_real, d_img,
                                tabER, tabEI, tabRR, tabRI)
    return _ce(scores)[0, 0]
